# Initial kernel scaffold; baseline (speedup 1.0000x reference)
import jax
import jax.numpy as jnp
from jax.experimental import pallas as pl


def kernel(x_gene, x_disease, ei_gg, ei_gd, ei_dg, Wp_g, bp_g, Wp_d, bp_d, ls_gg, ld_gg, ls_gd, ld_gd, ls_dg, ld_dg, kW, kb, q, linW, linb):
    raise NotImplementedError("write your pallas kernel here")


# speedup vs baseline: 21.5167x; 21.5167x over previous
import jax
import jax.numpy as jnp
from jax import lax
from jax.experimental import pallas as pl
from jax.experimental.pallas import tpu as pltpu
from jax.experimental.pallas import tpu_sc as plsc

N_GENE = 50000
N_DIS = 10000
E_GG = 320000
E_DG = 128000
IN = 128
HID = 128
HEADS = 8
DH = 16
OUT = 16
NSLAB = 4
SLAB = 32
NC = 2
NS = 16
NW = NC * NS
CH = 80
BLK = 1000
EPS = 1e-16


def _proj_a3_kernel(x_ref, w_ref, b_ref, o1_ref, o2_ref, o3_ref):
    a = jnp.dot(x_ref[...], w_ref[...], preferred_element_type=jnp.float32)
    a = a + b_ref[...]
    o1_ref[...] = a[:, 0:16]
    o2_ref[...] = a[:, 16:32]
    o3_ref[...] = a[:, 32:48]


def _proj_a1_kernel(x_ref, w_ref, b_ref, o1_ref):
    a = jnp.dot(x_ref[...], w_ref[...], preferred_element_type=jnp.float32)
    o1_ref[...] = a + b_ref[...]


def _proj_h_kernel(x_ref, w_ref, b_ref, h_ref):
    h = jnp.dot(x_ref[...], w_ref[...], preferred_element_type=jnp.float32)
    h = h + b_ref[...]
    for c in range(NSLAB):
        h_ref[c, :, :] = h[:, SLAB * c:SLAB * (c + 1)]


def _zero_units(s, zb, accs, nrows):
    nunit = nrows // 1000
    for i in range((nunit + NS - 1) // NS):
        u = i * NS + s

        @pl.when(u < nunit)
        def _():
            @pl.loop(0, 5)
            def _z(j):
                for acc in accs:
                    pltpu.sync_copy(
                        zb, acc.at[pl.ds(pl.multiple_of(u * 1000 + j * 200, 8), 200)])


def _dump_units(s, bnc, pairs, nrows):
    nunit = nrows // 1000
    for i in range((nunit + NS - 1) // NS):
        u = i * NS + s

        @pl.when(u < nunit)
        def _():
            for acc, out, off in pairs:
                @pl.loop(0, 5)
                def _d(j):
                    r0 = pl.multiple_of(u * 1000 + j * 200, 8)
                    pltpu.sync_copy(acc.at[pl.ds(r0, 200)], bnc)
                    r1 = pl.multiple_of(off + u * 1000 + j * 200, 8)
                    pltpu.sync_copy(bnc, out.at[pl.ds(r1, 200)])


def _sc_edge_e(src_gg, dst_gg, src_dg, dst_dg, as_gg, ad_gg, as_dg, ad_dg,
               e_gg, e_dg, den_gg, den_dg,
               sbuf, dbuf, asr, adr, erows, zb, bnc, dga, dda):
    c = lax.axis_index("c")
    s = lax.axis_index("s")
    w = c * NS + s
    zv = jnp.zeros((16,), jnp.float32)

    @pl.loop(0, 200)
    def _zzb(i):
        zb[i, :] = zv

    _zero_units(s, zb, [dga, dda], N_GENE)
    plsc.subcore_barrier()

    def do_edges(srcr, dstr, s_tab, d_tab, e_out, acc, n_edge):
        per = n_edge // NW
        nch = per // CH
        base0 = w * per

        @pl.loop(0, nch)
        def _chunk(k):
            base = pl.multiple_of(base0 + k * CH, 16)
            pltpu.sync_copy(srcr.at[pl.ds(base, CH)], sbuf)
            pltpu.sync_copy(dstr.at[pl.ds(base, CH)], dbuf)
            pltpu.sync_copy(s_tab.at[sbuf], asr)
            pltpu.sync_copy(d_tab.at[dbuf], adr)

            @pl.loop(0, CH)
            def _edge(i):
                xrow = asr[i, :] + adr[i, :]
                erows[i, :] = jnp.exp(jnp.maximum(xrow, 0.2 * xrow))

            pltpu.sync_copy(erows, acc.at[dbuf], add=True)
            pltpu.sync_copy(erows, e_out.at[pl.ds(base, CH)])

    do_edges(src_gg, dst_gg, as_gg, ad_gg, e_gg, dga, E_GG)
    do_edges(src_dg, dst_dg, as_dg, ad_dg, e_dg, dda, E_DG)
    plsc.subcore_barrier()

    _dump_units(s, bnc, [(dga, den_gg, c * N_GENE), (dda, den_dg, c * N_GENE)],
                N_GENE)


def _sc_edge_agg(src_gg, dst_gg, src_dg, dst_dg, hg_tab, hd_tab, e_gg, e_dg,
                 u_gg, u_dg,
                 sbuf, dbuf, ibuf, hrows, ebuf, zb, bnc, acc):
    c = lax.axis_index("c")
    s = lax.axis_index("s")
    zv = jnp.zeros((16,), jnp.float32)

    @pl.loop(0, 200)
    def _zzb(i):
        zb[i, pl.ds(0, 16)] = zv
        zb[i, pl.ds(16, 16)] = zv

    def one_round(srcr, dstr, htab, e_rows, u_out, n_edge, n_src, s_local):
        slab = 2 * c + s_local
        h_lo = jnp.full((16,), 0, jnp.int32) + 2 * slab
        h_hi = h_lo + 1

        _zero_units(s, zb, [acc], N_GENE)
        plsc.subcore_barrier()

        per = n_edge // NS
        nch = per // CH
        base0 = s * per
        off_base = slab * n_src

        @pl.loop(0, nch)
        def _chunk(k):
            base = pl.multiple_of(base0 + k * CH, 16)
            pltpu.sync_copy(srcr.at[pl.ds(base, CH)], sbuf)
            pltpu.sync_copy(dstr.at[pl.ds(base, CH)], dbuf)

            @pl.loop(0, CH // 16)
            def _mkidx(g):
                ibuf[pl.ds(g * 16, 16)] = sbuf[pl.ds(g * 16, 16)] + off_base

            pltpu.sync_copy(htab.at[ibuf], hrows)
            pltpu.sync_copy(e_rows.at[pl.ds(base, CH)], ebuf)

            @pl.loop(0, CH)
            def _edge(i):
                er = ebuf[i, :]
                e_lo = jnp.take(er, h_lo)
                e_hi = jnp.take(er, h_hi)
                hrows[i, pl.ds(0, 16)] = hrows[i, pl.ds(0, 16)] * e_lo
                hrows[i, pl.ds(16, 16)] = hrows[i, pl.ds(16, 16)] * e_hi

            pltpu.sync_copy(hrows, acc.at[dbuf], add=True)

        plsc.subcore_barrier()
        _dump_units(s, bnc, [(acc, u_out, slab * N_GENE)], N_GENE)
        plsc.subcore_barrier()

    one_round(src_gg, dst_gg, hg_tab, e_gg, u_gg, E_GG, N_GENE, 0)
    one_round(src_gg, dst_gg, hg_tab, e_gg, u_gg, E_GG, N_GENE, 1)
    one_round(src_dg, dst_dg, hd_tab, e_dg, u_dg, E_DG, N_DIS, 0)
    one_round(src_dg, dst_dg, hd_tab, e_dg, u_dg, E_DG, N_DIS, 1)


def _head_expand():
    col = lax.broadcasted_iota(jnp.int32, (2, SLAB), 1) // DH
    row = lax.broadcasted_iota(jnp.int32, (2, SLAB), 0)
    return (col == row).astype(jnp.float32)


def _metapath_o_slab(u_ref, dinv, e2m, c):
    dx = jnp.dot(dinv[:, 2 * c:2 * c + 2], e2m,
                 preferred_element_type=jnp.float32)
    return jnp.maximum(u_ref[c, :, :] * dx, 0.0)


def _dinv(d_ref):
    d = d_ref[0, :, 0:8] + d_ref[1, :, 0:8]
    return 1.0 / (d + EPS)


def _tc2_kernel(ugg_ref, dgg_ref, udg_ref, ddg_ref, kw_ref, kb_ref, t_ref):
    i = pl.program_id(0)

    @pl.when(i == 0)
    def _():
        t_ref[...] = jnp.zeros_like(t_ref)

    e2m = _head_expand()

    def metapath(u_ref, d_ref):
        dinv = _dinv(d_ref)
        accm = jnp.zeros((BLK, HID), jnp.float32)
        for c in range(NSLAB):
            o = _metapath_o_slab(u_ref, dinv, e2m, c)
            accm = accm + jnp.dot(o, kw_ref[SLAB * c:SLAB * (c + 1), :],
                                  preferred_element_type=jnp.float32)
        th = jnp.tanh(accm + kb_ref[...])
        return jnp.sum(th, axis=0, keepdims=True)

    t_ref[0:1, :] += metapath(ugg_ref, dgg_ref)
    t_ref[1:2, :] += metapath(udg_ref, ddg_ref)


def _tc3_kernel(ugg_ref, dgg_ref, udg_ref, ddg_ref, attn_ref, lw_ref, lb_ref,
                y_ref):
    a0 = attn_ref[0, 0]
    a1 = attn_ref[0, 1]
    e2m = _head_expand()
    dinv_gg = _dinv(dgg_ref)
    dinv_dg = _dinv(ddg_ref)
    acc = jnp.zeros((BLK, OUT), jnp.float32)
    for c in range(NSLAB):
        o_gg = _metapath_o_slab(ugg_ref, dinv_gg, e2m, c)
        o_dg = _metapath_o_slab(udg_ref, dinv_dg, e2m, c)
        comb = a0 * o_gg + a1 * o_dg
        acc = acc + jnp.dot(comb, lw_ref[SLAB * c:SLAB * (c + 1), :],
                            preferred_element_type=jnp.float32)
    y_ref[...] = acc + lb_ref[...]


def _pad_l(ls):
    l8 = ls[0][:, :, None] * jnp.eye(HEADS, dtype=jnp.float32)[:, None, :]
    l8 = jnp.pad(l8, ((0, 0), (0, 0), (0, 16 - HEADS)))
    return l8.reshape(HID, 16)


def kernel(x_gene, x_disease, ei_gg, ei_gd, ei_dg, Wp_g, bp_g, Wp_d, bp_d,
           ls_gg, ld_gg, ls_gd, ld_gd, ls_dg, ld_dg, kW, kb, q, linW, linb):
    f32 = jnp.float32
    src_gg, dst_gg = ei_gg[0], ei_gg[1]
    src_dg, dst_dg = ei_dg[0], ei_dg[1]

    l_s_gg, l_d_gg = _pad_l(ls_gg), _pad_l(ld_gg)
    l_s_dg, l_d_dg = _pad_l(ls_dg), _pad_l(ld_dg)
    wa_g = jnp.concatenate([Wp_g @ l_s_gg, Wp_g @ l_d_gg, Wp_g @ l_d_dg], 1)
    ba_g = jnp.concatenate([bp_g @ l_s_gg, bp_g @ l_d_gg, bp_g @ l_d_dg])
    wa_d = Wp_d @ l_s_dg
    ba_d = (bp_d @ l_s_dg).reshape(1, 16)
    ba_g = ba_g.reshape(1, 48)

    ng, nd = N_GENE // BLK, N_DIS // BLK

    full = lambda shp: pl.BlockSpec(shp, lambda i: (0,) * len(shp))
    rowblk = lambda w: pl.BlockSpec((BLK, w), lambda i: (i, 0))
    as_gg, ad_gg, ad_dg = pl.pallas_call(
        _proj_a3_kernel,
        grid=(ng,),
        in_specs=[rowblk(IN), full((IN, 48)), full((1, 48))],
        out_specs=[rowblk(16)] * 3,
        out_shape=[jax.ShapeDtypeStruct((N_GENE, 16), f32)] * 3,
    )(x_gene, wa_g, ba_g)
    as_dg = pl.pallas_call(
        _proj_a1_kernel,
        grid=(nd,),
        in_specs=[rowblk(IN), full((IN, 16)), full((1, 16))],
        out_specs=rowblk(16),
        out_shape=jax.ShapeDtypeStruct((N_DIS, 16), f32),
    )(x_disease, wa_d, ba_d)

    def proj_h(x, wp, bp, n, nblk):
        return pl.pallas_call(
            _proj_h_kernel,
            grid=(nblk,),
            in_specs=[rowblk(IN), full((IN, HID)), full((1, HID))],
            out_specs=pl.BlockSpec((NSLAB, BLK, SLAB), lambda i: (0, i, 0)),
            out_shape=jax.ShapeDtypeStruct((NSLAB, n, SLAB), f32),
        )(x, wp, bp.reshape(1, HID))

    hg_tab = proj_h(x_gene, Wp_g, bp_g, N_GENE, ng).reshape(NSLAB * N_GENE, SLAB)
    hd_tab = proj_h(x_disease, Wp_d, bp_d, N_DIS, nd).reshape(NSLAB * N_DIS, SLAB)

    mesh = plsc.VectorSubcoreMesh(core_axis_name="c", subcore_axis_name="s")
    sc_params = pltpu.CompilerParams(use_tc_tiling_on_sc=False)

    e_gg, e_dg, den_gg, den_dg = pl.kernel(
        _sc_edge_e,
        out_type=(
            jax.ShapeDtypeStruct((E_GG, 16), f32),
            jax.ShapeDtypeStruct((E_DG, 16), f32),
            jax.ShapeDtypeStruct((NC * N_GENE, 16), f32),
            jax.ShapeDtypeStruct((NC * N_GENE, 16), f32),
        ),
        mesh=mesh,
        compiler_params=sc_params,
        scratch_types=[
            pltpu.VMEM((CH,), jnp.int32),
            pltpu.VMEM((CH,), jnp.int32),
            pltpu.VMEM((CH, 16), f32),
            pltpu.VMEM((CH, 16), f32),
            pltpu.VMEM((CH, 16), f32),
            pltpu.VMEM((200, 16), f32),
            pltpu.VMEM((200, 16), f32),
            pltpu.VMEM_SHARED((N_GENE, 16), f32),
            pltpu.VMEM_SHARED((N_GENE, 16), f32),
        ],
    )(src_gg, dst_gg, src_dg, dst_dg, as_gg, ad_gg, as_dg, ad_dg)

    u_gg, u_dg = pl.kernel(
        _sc_edge_agg,
        out_type=(
            jax.ShapeDtypeStruct((NSLAB * N_GENE, SLAB), f32),
            jax.ShapeDtypeStruct((NSLAB * N_GENE, SLAB), f32),
        ),
        mesh=mesh,
        compiler_params=sc_params,
        scratch_types=[
            pltpu.VMEM((CH,), jnp.int32),
            pltpu.VMEM((CH,), jnp.int32),
            pltpu.VMEM((CH,), jnp.int32),
            pltpu.VMEM((CH, SLAB), f32),
            pltpu.VMEM((CH, 16), f32),
            pltpu.VMEM((200, SLAB), f32),
            pltpu.VMEM((200, SLAB), f32),
            pltpu.VMEM_SHARED((N_GENE, SLAB), f32),
        ],
    )(src_gg, dst_gg, src_dg, dst_dg, hg_tab, hd_tab, e_gg, e_dg)

    u_gg4 = u_gg.reshape(NSLAB, N_GENE, SLAB)
    u_dg4 = u_dg.reshape(NSLAB, N_GENE, SLAB)
    den_gg2 = den_gg.reshape(NC, N_GENE, 16)
    den_dg2 = den_dg.reshape(NC, N_GENE, 16)

    ublk = pl.BlockSpec((NSLAB, BLK, SLAB), lambda i: (0, i, 0))
    dblk = pl.BlockSpec((NC, BLK, 16), lambda i: (0, i, 0))
    t = pl.pallas_call(
        _tc2_kernel,
        grid=(ng,),
        in_specs=[ublk, dblk, ublk, dblk, full((HID, HID)), full((1, HID))],
        out_specs=pl.BlockSpec((2, HID), lambda i: (0, 0)),
        out_shape=jax.ShapeDtypeStruct((2, HID), f32),
    )(u_gg4, den_gg2, u_dg4, den_dg2, kW, kb.reshape(1, HID))

    score = jnp.sum((t / N_GENE) * q, axis=-1)
    attn = jax.nn.softmax(score).reshape(1, 2)

    y = pl.pallas_call(
        _tc3_kernel,
        grid=(ng,),
        in_specs=[ublk, dblk, ublk, dblk,
                  pl.BlockSpec(memory_space=pltpu.SMEM),
                  full((HID, OUT)), full((1, OUT))],
        out_specs=rowblk(OUT),
        out_shape=jax.ShapeDtypeStruct((N_GENE, OUT), f32),
    )(u_gg4, den_gg2, u_dg4, den_dg2, attn, linW, linb.reshape(1, OUT))
    return y
